# baseline (speedup 1.0000x reference)
import jax
import jax.numpy as jnp
from jax import lax
from jax.experimental import pallas as pl
from jax.experimental.pallas import tpu as pltpu
from jax.experimental.pallas import tpu_sc as plsc

_P = 64
_L = 16
_D = 4096
_K = 8
_B = 4

_NC = 2
_NS = 16
_NW = _NC * _NS

_NU = 4
_UL = _L // _NU
_CA = 1
_CB = _NU - _CA


def _select_body(cls_ref, key_ref, idx_ref):
    cls = cls_ref[...]
    key = key_ref[...]
    kn = key * lax.rsqrt(jnp.maximum(jnp.sum(key * key, axis=1, keepdims=True), 1e-12))
    cn = cls * lax.rsqrt(jnp.maximum(jnp.sum(cls * cls, axis=1, keepdims=True), 1e-12))
    sim = lax.dot_general(cn, kn, (((1,), (1,)), ((), ())),
                          preferred_element_type=jnp.float32)

    col = lax.broadcasted_iota(jnp.int32, (_B, _P), 1)
    kcol = lax.broadcasted_iota(jnp.int32, (_B, _K), 1)
    s = sim
    idxmat = jnp.zeros((_B, _K), jnp.float32)
    for k in range(_K):
        m = jnp.max(s, axis=1, keepdims=True)
        cand = jnp.where(s == m, col, _P)
        p = jnp.min(cand, axis=1, keepdims=True)
        idxmat = jnp.where(kcol == k, p.astype(jnp.float32), idxmat)
        s = jnp.where(col == p, -jnp.inf, s)

    rowi = lax.broadcasted_iota(jnp.int32, (_B * _K, _K), 0)
    ksel = (lax.broadcasted_iota(jnp.int32, (_B * _K, _K), 1)
            == lax.rem(rowi, _K)).astype(jnp.float32)
    bi = lax.broadcasted_iota(jnp.int32, (_B * _K, _B), 0)
    bsel = (lax.div(bi, _K) == lax.broadcasted_iota(jnp.int32, (_B * _K, _B), 1)
            ).astype(jnp.float32)
    idx_rows = lax.dot_general(bsel, idxmat, (((1,), (0,)), ((), ())),
                               preferred_element_type=jnp.float32)
    idx_flat = jnp.sum(idx_rows * ksel, axis=1, keepdims=True)
    idx_i = idx_flat.astype(jnp.int32)
    lane = lax.broadcasted_iota(jnp.int32, (_B * _K, 2 * 16), 1)
    grp = lax.div(lane, 16)
    off = jnp.where(grp == 0, 0, 1 + jnp.minimum(lax.rem(lane, 16), _CB - 1))
    idx_ref[...] = _NU * idx_i + off


def _simout_body(cls_ref, key_ref, idx_ref, sim_out_ref):
    cls = cls_ref[...]
    key = key_ref[...]
    kn = key * lax.rsqrt(jnp.maximum(jnp.sum(key * key, axis=1, keepdims=True), 1e-12))
    cn = cls * lax.rsqrt(jnp.maximum(jnp.sum(cls * cls, axis=1, keepdims=True), 1e-12))
    idx_i = lax.div(idx_ref[...], _NU)
    oh = (lax.broadcasted_iota(jnp.int32, (_B * _K, _P), 1)
          == idx_i[:, 0:1]).astype(jnp.float32)
    rows = lax.dot_general(oh, kn, (((1,), (0,)), ((), ())),
                           preferred_element_type=jnp.float32)
    bi = lax.broadcasted_iota(jnp.int32, (_B * _K, _B), 0)
    bsel = (lax.div(bi, _K) == lax.broadcasted_iota(jnp.int32, (_B * _K, _B), 1)
            ).astype(jnp.float32)
    cnrep = lax.dot_general(bsel, cn, (((1,), (0,)), ((), ())),
                            preferred_element_type=jnp.float32)
    sim_out_ref[...] = rows * cnrep


def _sc_gather_body(idx_hbm, prompt_hbm, out_hbm, idxv, buf_a, buf_b, sa, sb,
                    soa, sob):
    wid = lax.axis_index("s") * _NC + lax.axis_index("c")
    pltpu.sync_copy(idx_hbm.at[wid], idxv)
    ga = pltpu.async_copy(prompt_hbm.at[idxv.at[pl.ds(0, _CA)]], buf_a, sa)
    gb = pltpu.async_copy(prompt_hbm.at[idxv.at[pl.ds(16, _CB)]], buf_b, sb)
    ga.wait()
    oa = pltpu.async_copy(buf_a, out_hbm.at[pl.ds(wid * _NU, _CA)], soa)
    gb.wait()
    ob = pltpu.async_copy(buf_b, out_hbm.at[pl.ds(wid * _NU + _CA, _CB)], sob)
    oa.wait()
    ob.wait()


def kernel(x_embed, cls_features, prompt, prompt_key):
    del x_embed
    idx_rep = pl.pallas_call(
        _select_body,
        out_shape=jax.ShapeDtypeStruct((_B * _K, 2 * 16), jnp.int32),
    )(cls_features, prompt_key)

    mesh = plsc.VectorSubcoreMesh(core_axis_name="c", subcore_axis_name="s",
                                  num_cores=_NC, num_subcores=_NS)
    sc_gather = pl.kernel(
        _sc_gather_body,
        out_type=jax.ShapeDtypeStruct((_B * _K * _NU, _UL, _D), jnp.float32),
        mesh=mesh,
        scratch_types=[
            pltpu.VMEM((2 * 16,), jnp.int32),
            pltpu.VMEM((_CA, _UL, _D), jnp.float32),
            pltpu.VMEM((_CB, _UL, _D), jnp.float32),
            pltpu.SemaphoreType.DMA,
            pltpu.SemaphoreType.DMA,
            pltpu.SemaphoreType.DMA,
            pltpu.SemaphoreType.DMA,
        ],
    )
    out_units = sc_gather(idx_rep, prompt.reshape(_P * _NU, _UL, _D))

    sim_out = pl.pallas_call(
        _simout_body,
        out_shape=jax.ShapeDtypeStruct((_B * _K, _D), jnp.float32),
    )(cls_features, prompt_key, idx_rep)

    return (out_units.reshape(_B, _K * _L, _D), sim_out.reshape(_B, _K, _D))


# speedup vs baseline: 2.0536x; 2.0536x over previous
import jax
import jax.numpy as jnp
from jax import lax
from jax.experimental import pallas as pl
from jax.experimental.pallas import tpu as pltpu
from jax.experimental.pallas import tpu_sc as plsc

_P = 64
_L = 16
_D = 4096
_K = 8
_B = 4

_NC = 2
_NS = 16
_NW = _NC * _NS

_NU = 2
_UL = _L // _NU
_CA = 1
_CB = _NU - _CA


def _select_body(cls_ref, key_ref, idx_ref):
    cls = cls_ref[...]
    key = key_ref[...]
    kn = key * lax.rsqrt(jnp.maximum(jnp.sum(key * key, axis=1, keepdims=True), 1e-12))
    cn = cls * lax.rsqrt(jnp.maximum(jnp.sum(cls * cls, axis=1, keepdims=True), 1e-12))
    sim = lax.dot_general(cn, kn, (((1,), (1,)), ((), ())),
                          preferred_element_type=jnp.float32)

    col = lax.broadcasted_iota(jnp.int32, (_B, _P), 1)
    kcol = lax.broadcasted_iota(jnp.int32, (_B, _K), 1)
    s = sim
    idxmat = jnp.zeros((_B, _K), jnp.float32)
    for k in range(_K):
        m = jnp.max(s, axis=1, keepdims=True)
        cand = jnp.where(s == m, col, _P)
        p = jnp.min(cand, axis=1, keepdims=True)
        idxmat = jnp.where(kcol == k, p.astype(jnp.float32), idxmat)
        s = jnp.where(col == p, -jnp.inf, s)

    rowi = lax.broadcasted_iota(jnp.int32, (_B * _K, _K), 0)
    ksel = (lax.broadcasted_iota(jnp.int32, (_B * _K, _K), 1)
            == lax.rem(rowi, _K)).astype(jnp.float32)
    bi = lax.broadcasted_iota(jnp.int32, (_B * _K, _B), 0)
    bsel = (lax.div(bi, _K) == lax.broadcasted_iota(jnp.int32, (_B * _K, _B), 1)
            ).astype(jnp.float32)
    idx_rows = lax.dot_general(bsel, idxmat, (((1,), (0,)), ((), ())),
                               preferred_element_type=jnp.float32)
    idx_flat = jnp.sum(idx_rows * ksel, axis=1, keepdims=True)
    idx_i = idx_flat.astype(jnp.int32)
    lane = lax.broadcasted_iota(jnp.int32, (_B * _K, 2 * 16), 1)
    grp = lax.div(lane, 16)
    off = jnp.where(grp == 0, 0, 1 + jnp.minimum(lax.rem(lane, 16), _CB - 1))
    idx_ref[...] = _NU * idx_i + off


def _simout_body(cls_ref, key_ref, idx_ref, sim_out_ref):
    cls = cls_ref[...]
    key = key_ref[...]
    kn = key * lax.rsqrt(jnp.maximum(jnp.sum(key * key, axis=1, keepdims=True), 1e-12))
    cn = cls * lax.rsqrt(jnp.maximum(jnp.sum(cls * cls, axis=1, keepdims=True), 1e-12))
    idx_i = lax.div(idx_ref[...], _NU)
    oh = (lax.broadcasted_iota(jnp.int32, (_B * _K, _P), 1)
          == idx_i[:, 0:1]).astype(jnp.float32)
    rows = lax.dot_general(oh, kn, (((1,), (0,)), ((), ())),
                           preferred_element_type=jnp.float32)
    bi = lax.broadcasted_iota(jnp.int32, (_B * _K, _B), 0)
    bsel = (lax.div(bi, _K) == lax.broadcasted_iota(jnp.int32, (_B * _K, _B), 1)
            ).astype(jnp.float32)
    cnrep = lax.dot_general(bsel, cn, (((1,), (0,)), ((), ())),
                            preferred_element_type=jnp.float32)
    sim_out_ref[...] = rows * cnrep


def _sc_gather_body(idx_hbm, prompt_hbm, out_hbm, idxv, buf_a, buf_b, sa, sb,
                    soa, sob):
    wid = lax.axis_index("s") * _NC + lax.axis_index("c")
    pltpu.sync_copy(idx_hbm.at[wid], idxv)
    ga = pltpu.async_copy(prompt_hbm.at[idxv.at[pl.ds(0, _CA)]], buf_a, sa)
    gb = pltpu.async_copy(prompt_hbm.at[idxv.at[pl.ds(16, _CB)]], buf_b, sb)
    ga.wait()
    oa = pltpu.async_copy(buf_a, out_hbm.at[pl.ds(wid * _NU, _CA)], soa)
    gb.wait()
    ob = pltpu.async_copy(buf_b, out_hbm.at[pl.ds(wid * _NU + _CA, _CB)], sob)
    oa.wait()
    ob.wait()


def kernel(x_embed, cls_features, prompt, prompt_key):
    del x_embed
    idx_rep = pl.pallas_call(
        _select_body,
        out_shape=jax.ShapeDtypeStruct((_B * _K, 2 * 16), jnp.int32),
    )(cls_features, prompt_key)

    mesh = plsc.VectorSubcoreMesh(core_axis_name="c", subcore_axis_name="s",
                                  num_cores=_NC, num_subcores=_NS)
    sc_gather = pl.kernel(
        _sc_gather_body,
        out_type=jax.ShapeDtypeStruct((_B * _K * _NU, _UL, _D), jnp.float32),
        mesh=mesh,
        scratch_types=[
            pltpu.VMEM((2 * 16,), jnp.int32),
            pltpu.VMEM((_CA, _UL, _D), jnp.float32),
            pltpu.VMEM((_CB, _UL, _D), jnp.float32),
            pltpu.SemaphoreType.DMA,
            pltpu.SemaphoreType.DMA,
            pltpu.SemaphoreType.DMA,
            pltpu.SemaphoreType.DMA,
        ],
    )
    out_units = sc_gather(idx_rep, prompt.reshape(_P * _NU, _UL, _D))

    sim_out = pl.pallas_call(
        _simout_body,
        out_shape=jax.ShapeDtypeStruct((_B * _K, _D), jnp.float32),
    )(cls_features, prompt_key, idx_rep)

    return (out_units.reshape(_B, _K * _L, _D), sim_out.reshape(_B, _K, _D))
